# baseline (speedup 1.0000x reference)
import functools
import math

import jax
import jax.numpy as jnp
from jax import lax
from jax.experimental import pallas as pl
from jax.experimental.pallas import tpu as pltpu
from jax.experimental.pallas import tpu_sc as plsc

D_MODEL = 64
SCALE = math.sqrt(D_MODEL)
LANES = 16
NUM_CORES = 2
NUM_SUBCORES = 16
NUM_WORKERS = NUM_CORES * NUM_SUBCORES
BW = 128


@functools.lru_cache(maxsize=None)
def _build(S0, S1):
    n_chunks = BW // LANES

    mesh = plsc.VectorSubcoreMesh(
        core_axis_name="c", subcore_axis_name="s",
        num_cores=NUM_CORES, num_subcores=NUM_SUBCORES)

    @functools.partial(
        pl.kernel,
        mesh=mesh,
        out_type=jax.ShapeDtypeStruct((S1, D_MODEL, S0), jnp.float32),
        scratch_types=[
            pltpu.VMEM((S1, BW), jnp.int32),
            pltpu.VMEM((BW,), jnp.int32),
            pltpu.VMEM((BW,), jnp.int32),
            pltpu.VMEM((BW, 2 * D_MODEL), jnp.float32),
            pltpu.VMEM((BW, 2 * D_MODEL), jnp.float32),
            pltpu.VMEM((D_MODEL, BW), jnp.float32),
            pltpu.SemaphoreType.DMA,
            pltpu.SemaphoreType.DMA,
        ],
        compiler_params=pltpu.CompilerParams(needs_layout_passes=False),
    )
    def emb(xt_hbm, tab2_hbm, out_hbm, idx_all, qa, qb, rowsa, rowsb,
            outt, sema, semb):
        wid = lax.axis_index("s") * NUM_CORES + lax.axis_index("c")
        r0 = wid * BW

        pltpu.sync_copy(xt_hbm.at[:, pl.ds(r0, BW)], idx_all)

        def fire(j, q_v, rows_v, sem):
            for c in range(n_chunks):
                sl = pl.ds(c * LANES, LANES)
                q_v[sl] = lax.shift_right_logical(idx_all[j, sl], 1)
            pltpu.async_copy(tab2_hbm.at[q_v], rows_v, sem)

        def process(j, q_v, rows_v, sem):
            pltpu.make_async_copy(tab2_hbm.at[q_v], rows_v, sem).wait()
            for c in range(n_chunks):
                sl = pl.ds(c * LANES, LANES)
                row_c = jnp.arange(LANES, dtype=jnp.int32) + (c * LANES)
                off_c = (idx_all[j, sl] & 1) << 6

                def kbody(k, carry, row_c=row_c, off_c=off_c, sl=sl):
                    col = off_c + k
                    v = plsc.load_gather(rows_v, [row_c, col])
                    outt[k, sl] = v * SCALE
                    return carry

                lax.fori_loop(0, D_MODEL, kbody, 0)
            pltpu.sync_copy(outt, out_hbm.at[j, :, pl.ds(r0, BW)])

        fire(0, qa, rowsa, sema)

        def pair_body(jj, carry):
            j0 = 2 * jj
            j1 = j0 + 1
            fire(j1, qb, rowsb, semb)
            process(j0, qa, rowsa, sema)

            @pl.when(j1 + 1 < S1)
            def _():
                fire(j1 + 1, qa, rowsa, sema)

            process(j1, qb, rowsb, semb)
            return carry

        lax.fori_loop(0, S1 // 2, pair_body, 0)

    return emb


def kernel(x, table):
    S0, S1 = x.shape
    xt = x.T.astype(jnp.int32)
    tab2 = table.reshape(table.shape[0] // 2, 2 * D_MODEL)
    out = _build(S0, S1)(xt, tab2)
    return out.transpose(2, 0, 1)
